# Initial kernel scaffold; baseline (speedup 1.0000x reference)
import jax
import jax.numpy as jnp
from jax.experimental import pallas as pl


def kernel(sparse_idx, dense_feats, seq_idx, table_sparse, table_seq, W1, b1, W2, b2):
    raise NotImplementedError("write your pallas kernel here")


# speedup vs baseline: 9.4122x; 9.4122x over previous
import functools

import jax
import jax.numpy as jnp
from jax import lax
from jax.experimental import pallas as pl
from jax.experimental.pallas import tpu as pltpu
from jax.experimental.pallas import tpu_sc as plsc

_D = 16
_B = 16384
_NS = 26
_ND = 13
_L = 50
_U1 = 128
_U2 = 64

_NC = 2
_NSUB = 16
_NW = _NC * _NSUB
_BPW = _B // _NW
_CHUNK = 64
_NCHUNK = _BPW // _CHUNK
_SP_ROWS = _CHUNK * _NS
_SQ_ROWS = _CHUNK * _L
_IDXW = 128
_SP_STREAMS = _SP_ROWS // _IDXW
_SQ_STREAMS = _SQ_ROWS // _IDXW


def _sc_body(sp_idx_hbm, sq_idx_hbm, tsp_hbm, tsq_hbm,
             sp_out, seqsum_out,
             sp_idx_v, sp_rows_v, sq_idx_v, sq_rows_v, seqsum_v, sem):
    wid = lax.axis_index("s") * _NC + lax.axis_index("c")

    pltpu.sync_copy(
        sp_idx_hbm.at[pl.ds(wid * (_NCHUNK * _SP_STREAMS), _NCHUNK * _SP_STREAMS)],
        sp_idx_v)
    pltpu.sync_copy(
        sq_idx_hbm.at[pl.ds(wid * (_NCHUNK * _SQ_STREAMS), _NCHUNK * _SQ_STREAMS)],
        sq_idx_v)

    def sparse_chunk(c, carry):
        descs = [
            pltpu.async_copy(tsp_hbm.at[sp_idx_v.at[c * _SP_STREAMS + j]],
                             sp_rows_v.at[pl.ds(j * _IDXW, _IDXW)], sem)
            for j in range(_SP_STREAMS)
        ]
        for d in descs:
            d.wait()
        row0 = (wid * _BPW + c * _CHUNK) * _NS
        pltpu.sync_copy(sp_rows_v, sp_out.at[pl.ds(row0, _SP_ROWS)])
        return carry

    lax.fori_loop(0, _NCHUNK, sparse_chunk, 0)

    def seq_chunk(c, carry):
        descs = [
            pltpu.async_copy(tsq_hbm.at[sq_idx_v.at[c * _SQ_STREAMS + j]],
                             sq_rows_v.at[pl.ds(j * _IDXW, _IDXW)], sem)
            for j in range(_SQ_STREAMS)
        ]
        for d in descs:
            d.wait()

        def item(i, carry2):
            base = i * _L
            accs = [sq_rows_v[base + j, :] for j in range(4)]
            for j in range(4, _L):
                accs[j % 4] = accs[j % 4] + sq_rows_v[base + j, :]
            seqsum_v[i, :] = (accs[0] + accs[1]) + (accs[2] + accs[3])
            return carry2

        lax.fori_loop(0, _CHUNK, item, 0)
        item0 = wid * _BPW + c * _CHUNK
        pltpu.sync_copy(seqsum_v, seqsum_out.at[pl.ds(item0, _CHUNK)])
        return carry

    lax.fori_loop(0, _NCHUNK, seq_chunk, 0)


def _make_sc_gather():
    return functools.partial(
        pl.kernel,
        mesh=plsc.VectorSubcoreMesh(core_axis_name="c", subcore_axis_name="s"),
        compiler_params=pltpu.CompilerParams(use_tc_tiling_on_sc=False),
        out_type=[
            jax.ShapeDtypeStruct((_B * _NS, _D), jnp.float32),
            jax.ShapeDtypeStruct((_B, _D), jnp.float32),
        ],
        scratch_types=[
            pltpu.VMEM((_NCHUNK * _SP_STREAMS, _IDXW), jnp.int32),
            pltpu.VMEM((_SP_ROWS, _D), jnp.float32),
            pltpu.VMEM((_NCHUNK * _SQ_STREAMS, _IDXW), jnp.int32),
            pltpu.VMEM((_SQ_ROWS, _D), jnp.float32),
            pltpu.VMEM((_CHUNK, _D), jnp.float32),
            pltpu.SemaphoreType.DMA,
        ],
    )(_sc_body)


_TC_R = 512


def _tc_body(sp_ref, dn_ref, ss_ref, sqi_ref, r0_ref,
             w1a_ref, w1b_ref, w1c_ref, b1_ref, w2_ref, b2_ref, out_ref):
    hi = lax.Precision.HIGHEST
    sq = sqi_ref[...]
    cnt = jnp.sum((sq != 0).astype(jnp.float32), axis=1, keepdims=True)
    mid = (ss_ref[...] - (float(_L) - cnt) * r0_ref[...]) / jnp.maximum(cnt, 1.0)
    nrm = jnp.sqrt(jnp.sum(mid * mid, axis=1, keepdims=True))
    mid = mid / jnp.maximum(nrm, 1e-12)
    h = (jnp.dot(sp_ref[...], w1a_ref[...], precision=hi)
         + jnp.dot(dn_ref[...], w1b_ref[...], precision=hi)
         + jnp.dot(mid, w1c_ref[...], precision=hi)
         + b1_ref[...])
    h = jnp.maximum(h, 0.0)
    h2 = jnp.dot(h, w2_ref[...], precision=hi) + b2_ref[...]
    h2 = jnp.maximum(h2, 0.0)
    nrm2 = jnp.sqrt(jnp.sum(h2 * h2, axis=1, keepdims=True))
    out_ref[...] = h2 / jnp.maximum(nrm2, 1e-12)


def kernel(sparse_idx, dense_feats, seq_idx, table_sparse, table_seq,
           W1, b1, W2, b2):
    sp_idx2 = sparse_idx.astype(jnp.int32).reshape(_B * _NS // _IDXW, _IDXW)
    sq_idx2 = seq_idx.astype(jnp.int32).reshape(_B * _L // _IDXW, _IDXW)
    sp_rows, seqsum = _make_sc_gather()(sp_idx2, sq_idx2, table_sparse, table_seq)

    sp = sp_rows.reshape(_B, _NS * _D)
    dnp = jnp.pad(dense_feats, ((0, 0), (0, 3)))
    w1a = W1[: _NS * _D]
    w1b = jnp.pad(W1[_NS * _D: _NS * _D + _ND], ((0, 3), (0, 0)))
    w1c = W1[_NS * _D + _ND:]
    row0 = table_seq[0:1]

    nblk = _B // _TC_R
    out = pl.pallas_call(
        _tc_body,
        grid=(nblk,),
        in_specs=[
            pl.BlockSpec((_TC_R, _NS * _D), lambda i: (i, 0)),
            pl.BlockSpec((_TC_R, 16), lambda i: (i, 0)),
            pl.BlockSpec((_TC_R, _D), lambda i: (i, 0)),
            pl.BlockSpec((_TC_R, _L), lambda i: (i, 0)),
            pl.BlockSpec((1, _D), lambda i: (0, 0)),
            pl.BlockSpec((_NS * _D, _U1), lambda i: (0, 0)),
            pl.BlockSpec((16, _U1), lambda i: (0, 0)),
            pl.BlockSpec((_D, _U1), lambda i: (0, 0)),
            pl.BlockSpec((1, _U1), lambda i: (0, 0)),
            pl.BlockSpec((_U1, _U2), lambda i: (0, 0)),
            pl.BlockSpec((1, _U2), lambda i: (0, 0)),
        ],
        out_specs=pl.BlockSpec((_TC_R, _U2), lambda i: (i, 0)),
        out_shape=jax.ShapeDtypeStruct((_B, _U2), jnp.float32),
    )(sp, dnp, seqsum, seq_idx.astype(jnp.int32), row0,
      w1a, w1b, w1c, b1.reshape(1, _U1), W2, b2.reshape(1, _U2))
    return out
